# baseline (speedup 1.0000x reference)
import functools

import numpy as np
import jax
import jax.numpy as jnp
from jax.experimental import pallas as pl
from jax.experimental.pallas import tpu as pltpu

_TINY = np.float32(np.finfo(np.float32).tiny)
_BLK = 8192
_SLC = 256

_K0 = np.uint32(0)
_K1 = np.uint32(42)
_KS2 = np.uint32(np.uint32(0x1BD11BDA) ^ _K0 ^ _K1)
_ROT0 = (13, 15, 26, 6)
_ROT1 = (17, 29, 16, 24)


def _rotl(x, d):
    return (x << np.uint32(d)) | (x >> np.uint32(32 - d))


def _threefry_xor(c1):
    x0 = jnp.zeros_like(c1) + _K0
    x1 = c1 + _K1

    def rounds(x0, x1, rots):
        for r in rots:
            x0 = x0 + x1
            x1 = x0 ^ _rotl(x1, r)
        return x0, x1

    x0, x1 = rounds(x0, x1, _ROT0)
    x0 = x0 + _K1
    x1 = x1 + np.uint32(_KS2 + np.uint32(1))
    x0, x1 = rounds(x0, x1, _ROT1)
    x0 = x0 + _KS2
    x1 = x1 + np.uint32(_K0 + np.uint32(2))
    x0, x1 = rounds(x0, x1, _ROT0)
    x0 = x0 + _K0
    x1 = x1 + np.uint32(_K1 + np.uint32(3))
    x0, x1 = rounds(x0, x1, _ROT1)
    x0 = x0 + _K1
    x1 = x1 + np.uint32(_KS2 + np.uint32(4))
    x0, x1 = rounds(x0, x1, _ROT0)
    x0 = x0 + _KS2
    x1 = x1 + np.uint32(_K0 + np.uint32(5))
    return x0 ^ x1


def _sampler_kernel(logits_ref, out_ref, best_val, best_idx, *, nblk, V, B):
    k = pl.program_id(0)

    @pl.when(k == 0)
    def _init():
        best_val[...] = jnp.full((B, 1), -jnp.inf, jnp.float32)
        best_idx[...] = jnp.zeros((B, 1), jnp.int32)

    base_col = k * _BLK
    rowu = jax.lax.broadcasted_iota(jnp.uint32, (B, _SLC), 0) * np.uint32(V)
    coli = jax.lax.broadcasted_iota(jnp.int32, (B, _SLC), 1)

    def body(s, carry):
        bv, bi = carry
        col0 = s * _SLC
        gcol = coli + (base_col + col0)
        counts = rowu + gcol.astype(jnp.uint32)
        bits = _threefry_xor(counts)
        fb = (bits >> np.uint32(9)) | np.uint32(0x3F800000)
        f = jax.lax.bitcast_convert_type(fb, jnp.float32) - np.float32(1.0)
        u = jnp.maximum(_TINY, f * np.float32(np.float32(1.0) - _TINY) + _TINY)
        g = -jnp.log(-jnp.log(u))
        vals = logits_ref[:, pl.ds(col0, _SLC)] + g
        vals = jnp.where(gcol < V, vals, -jnp.inf)
        m = jnp.max(vals, axis=1, keepdims=True)
        idx = jnp.min(jnp.where(vals == m, gcol, np.int32(2**30)),
                      axis=1, keepdims=True)
        upd = m > bv
        return jnp.where(upd, m, bv), jnp.where(upd, idx, bi)

    bv, bi = jax.lax.fori_loop(
        0, _BLK // _SLC, body, (best_val[...], best_idx[...]))
    best_val[...] = bv
    best_idx[...] = bi

    @pl.when(k == nblk - 1)
    def _done():
        out_ref[...] = best_idx[...]


def kernel(logits):
    B, V = logits.shape
    nblk = (V + _BLK - 1) // _BLK
    return pl.pallas_call(
        functools.partial(_sampler_kernel, nblk=nblk, V=V, B=B),
        grid=(nblk,),
        in_specs=[pl.BlockSpec((B, _BLK), lambda k: (0, k))],
        out_specs=pl.BlockSpec((B, 1), lambda k: (0, 0)),
        out_shape=jax.ShapeDtypeStruct((B, 1), jnp.int32),
        scratch_shapes=[pltpu.VMEM((B, 1), jnp.float32),
                        pltpu.VMEM((B, 1), jnp.int32)],
    )(logits)


# speedup vs baseline: 2.7797x; 2.7797x over previous
import functools

import numpy as np
import jax
import jax.numpy as jnp
from jax.experimental import pallas as pl
from jax.experimental.pallas import tpu as pltpu

_TINY = np.float32(np.finfo(np.float32).tiny)
_BLK = 8192
_SLC = 256

_K0 = np.uint32(0)
_K1 = np.uint32(42)
_KS2 = np.uint32(np.uint32(0x1BD11BDA) ^ _K0 ^ _K1)
_ROT0 = (13, 15, 26, 6)
_ROT1 = (17, 29, 16, 24)


def _rotl(x, d):
    return (x << np.uint32(d)) | (x >> np.uint32(32 - d))


def _threefry_xor(x1):
    def rounds(x0, x1, rots):
        for r in rots:
            x0 = x0 + x1
            x1 = x0 ^ _rotl(x1, r)
        return x0, x1

    x0 = x1
    x1 = x0 ^ _rotl(x1, _ROT0[0])
    x0, x1 = rounds(x0, x1, _ROT0[1:])
    x0 = x0 + _K1
    x1 = x1 + np.uint32(_KS2 + np.uint32(1))
    x0, x1 = rounds(x0, x1, _ROT1)
    x0 = x0 + _KS2
    x1 = x1 + np.uint32(_K0 + np.uint32(2))
    x0, x1 = rounds(x0, x1, _ROT0)
    x0 = x0 + _K0
    x1 = x1 + np.uint32(_K1 + np.uint32(3))
    x0, x1 = rounds(x0, x1, _ROT1)
    x0 = x0 + _K1
    x1 = x1 + np.uint32(_KS2 + np.uint32(4))
    x0, x1 = rounds(x0, x1, _ROT0)
    x0 = x0 + _KS2
    x1 = x1 + np.uint32(_K0 + np.uint32(5))
    return x0 ^ x1


def _gumbel(bits):
    fb = (bits >> np.uint32(9)) | np.uint32(0x3F800000)
    f = jax.lax.bitcast_convert_type(fb, jnp.float32) - np.float32(1.0)
    u = jnp.maximum(_TINY, f * np.float32(np.float32(1.0) - _TINY) + _TINY)
    return -jnp.log(-jnp.log(u))


def _sampler_kernel(logits_ref, out_ref, vmax_ref, vidx_ref, *, nblk, V, B):
    k = pl.program_id(0)

    @pl.when(k == 0)
    def _init():
        vmax_ref[...] = jnp.full((B, _SLC), -jnp.inf, jnp.float32)
        vidx_ref[...] = jnp.zeros((B, _SLC), jnp.int32)

    nslc = _BLK // _SLC
    base_col = k * _BLK
    rv = (jax.lax.broadcasted_iota(jnp.uint32, (B, _SLC), 0) * np.uint32(V)
          + jax.lax.broadcasted_iota(jnp.uint32, (B, _SLC), 1))
    coli = jax.lax.broadcasted_iota(jnp.int32, (B, _SLC), 1)

    def body(s, carry):
        vmax, vidx = carry
        col0 = s * _SLC
        x1 = rv + (base_col + col0 + 42).astype(jnp.uint32)
        g = _gumbel(_threefry_xor(x1))
        vals = logits_ref[:, pl.ds(col0, _SLC)] + g
        vals = jnp.where(coli < V - (base_col + col0), vals, -jnp.inf)
        upd = vals > vmax
        sid = jnp.full((B, _SLC), 0, jnp.int32) + (k * nslc + s)
        return jnp.where(upd, vals, vmax), jnp.where(upd, sid, vidx)

    vmax, vidx = jax.lax.fori_loop(
        0, nslc, body, (vmax_ref[...], vidx_ref[...]))
    vmax_ref[...] = vmax
    vidx_ref[...] = vidx

    @pl.when(k == nblk - 1)
    def _done():
        vm = vmax_ref[...]
        m = jnp.max(vm, axis=1, keepdims=True)
        gidx = vidx_ref[...] * _SLC + coli
        out_ref[...] = jnp.min(
            jnp.where(vm == m, gidx, np.int32(2**30)), axis=1, keepdims=True)


def kernel(logits):
    B, V = logits.shape
    nblk = (V + _BLK - 1) // _BLK
    return pl.pallas_call(
        functools.partial(_sampler_kernel, nblk=nblk, V=V, B=B),
        grid=(nblk,),
        in_specs=[pl.BlockSpec((B, _BLK), lambda k: (0, k))],
        out_specs=pl.BlockSpec((B, 1), lambda k: (0, 0)),
        out_shape=jax.ShapeDtypeStruct((B, 1), jnp.int32),
        scratch_shapes=[pltpu.VMEM((B, _SLC), jnp.float32),
                        pltpu.VMEM((B, _SLC), jnp.int32)],
    )(logits)
